# speedup vs baseline: 1.6539x; 1.6539x over previous
import jax
import jax.numpy as jnp
from jax.experimental import pallas as pl
from jax.experimental.pallas import tpu as pltpu

BANK = 512
DIM = 768
EPSV = 1e-6
MOM = 0.9
BLK = 512
NBLK = (16384 - BANK) // BLK


def _norm_rows(x):
    n = jnp.sqrt(jnp.sum(x * x, axis=1, keepdims=True))
    return x / jnp.maximum(n, EPSV)


def _body(proto_ref, rem_ref, out_ref, pn_ref, sums_ref, counts_ref):
    i = pl.program_id(0)

    @pl.when(i == 0)
    def _init():
        pn = _norm_rows(_norm_rows(proto_ref[...]))
        pn_ref[...] = pn
        sums_ref[...] = jnp.zeros_like(sums_ref)
        counts_ref[...] = jnp.zeros_like(counts_ref)

    e = rem_ref[...]
    rn = _norm_rows(e)
    pn = pn_ref[...]
    s = jax.lax.dot_general(
        rn, pn, (((1,), (1,)), ((), ())),
        preferred_element_type=jnp.float32,
        precision=jax.lax.Precision.HIGHEST,
    )
    a = jnp.argmax(s, axis=1)
    onehot = (jax.lax.broadcasted_iota(jnp.int32, s.shape, 1)
              == a[:, None]).astype(jnp.float32)
    sums_ref[...] += jax.lax.dot_general(
        onehot, rn, (((0,), (0,)), ((), ())),
        preferred_element_type=jnp.float32,
        precision=jax.lax.Precision.HIGHEST,
    )
    counts_ref[...] += jax.lax.dot_general(
        onehot, jnp.ones((BLK, 1), jnp.float32), (((0,), (0,)), ((), ())),
        preferred_element_type=jnp.float32,
        precision=jax.lax.Precision.HIGHEST,
    )

    @pl.when(i == NBLK - 1)
    def _fin():
        counts = counts_ref[...]
        means = sums_ref[...] / jnp.maximum(counts, 1.0)
        pn = pn_ref[...]
        upd = MOM * pn + (1.0 - MOM) * means
        updn = _norm_rows(upd)
        out_ref[...] = jnp.where(counts > 0.0, updn, pn)


def kernel(embeddings):
    emb = embeddings.astype(jnp.float32)
    return pl.pallas_call(
        _body,
        grid=(NBLK,),
        in_specs=[
            pl.BlockSpec((BANK, DIM), lambda i: (0, 0)),
            pl.BlockSpec((BLK, DIM), lambda i: (i + 1, 0)),
        ],
        out_specs=pl.BlockSpec((BANK, DIM), lambda i: (0, 0)),
        out_shape=jax.ShapeDtypeStruct((BANK, DIM), jnp.float32),
        scratch_shapes=[
            pltpu.VMEM((BANK, DIM), jnp.float32),
            pltpu.VMEM((BANK, DIM), jnp.float32),
            pltpu.VMEM((BANK, 1), jnp.float32),
        ],
    )(emb, emb)


# speedup vs baseline: 5.2028x; 3.1458x over previous
import jax
import jax.numpy as jnp
from jax.experimental import pallas as pl
from jax.experimental.pallas import tpu as pltpu

BANK = 512
DIM = 768
EPSV = 1e-6
MOM = 0.9
BLK = 512
NBLK = (16384 - BANK) // BLK


def _norm_rows(x):
    n = jnp.sqrt(jnp.sum(x * x, axis=1, keepdims=True))
    return x / jnp.maximum(n, EPSV)


def _body(proto_ref, rem_ref, out_ref, pn_ref, sums_ref, counts_ref):
    i = pl.program_id(0)

    @pl.when(i == 0)
    def _init():
        pn = _norm_rows(_norm_rows(proto_ref[...]))
        pn_ref[...] = pn
        sums_ref[...] = jnp.zeros_like(sums_ref)
        counts_ref[...] = jnp.zeros_like(counts_ref)

    e = rem_ref[...]
    rn = _norm_rows(e)
    pn = pn_ref[...]
    s = jax.lax.dot_general(
        rn, pn, (((1,), (1,)), ((), ())),
        preferred_element_type=jnp.float32,
        precision=jax.lax.Precision.DEFAULT,
    )
    a = jnp.argmax(s, axis=1)
    onehot = (jax.lax.broadcasted_iota(jnp.int32, s.shape, 1)
              == a[:, None]).astype(jnp.float32)
    sums_ref[...] += jax.lax.dot_general(
        onehot, rn, (((0,), (0,)), ((), ())),
        preferred_element_type=jnp.float32,
        precision=jax.lax.Precision.DEFAULT,
    )
    counts_ref[...] += jax.lax.dot_general(
        onehot, jnp.ones((BLK, 1), jnp.float32), (((0,), (0,)), ((), ())),
        preferred_element_type=jnp.float32,
        precision=jax.lax.Precision.DEFAULT,
    )

    @pl.when(i == NBLK - 1)
    def _fin():
        counts = counts_ref[...]
        means = sums_ref[...] / jnp.maximum(counts, 1.0)
        pn = pn_ref[...]
        upd = MOM * pn + (1.0 - MOM) * means
        updn = _norm_rows(upd)
        out_ref[...] = jnp.where(counts > 0.0, updn, pn)


def kernel(embeddings):
    emb = embeddings.astype(jnp.float32)
    return pl.pallas_call(
        _body,
        grid=(NBLK,),
        in_specs=[
            pl.BlockSpec((BANK, DIM), lambda i: (0, 0)),
            pl.BlockSpec((BLK, DIM), lambda i: (i + 1, 0)),
        ],
        out_specs=pl.BlockSpec((BANK, DIM), lambda i: (0, 0)),
        out_shape=jax.ShapeDtypeStruct((BANK, DIM), jnp.float32),
        scratch_shapes=[
            pltpu.VMEM((BANK, DIM), jnp.float32),
            pltpu.VMEM((BANK, DIM), jnp.float32),
            pltpu.VMEM((BANK, 1), jnp.float32),
        ],
    )(emb, emb)


# speedup vs baseline: 5.2543x; 1.0099x over previous
import jax
import jax.numpy as jnp
from jax.experimental import pallas as pl
from jax.experimental.pallas import tpu as pltpu

BANK = 512
DIM = 768
EPSV = 1e-6
MOM = 0.9
BLK = 512
NBLK = (16384 - BANK) // BLK


def _norm_rows(x):
    n = jnp.sqrt(jnp.sum(x * x, axis=1, keepdims=True))
    return x / jnp.maximum(n, EPSV)


def _body(proto_ref, rem_ref, out_ref, pn_ref, sums_ref, counts_ref):
    i = pl.program_id(0)

    @pl.when(i == 0)
    def _init():
        pn = _norm_rows(_norm_rows(proto_ref[...]))
        pn_ref[...] = pn
        sums_ref[...] = jnp.zeros_like(sums_ref)
        counts_ref[...] = jnp.zeros_like(counts_ref)

    e = rem_ref[...]
    rn = _norm_rows(e)
    rnb = rn.astype(jnp.bfloat16)
    pnb = pn_ref[...].astype(jnp.bfloat16)
    s = jax.lax.dot_general(
        rnb, pnb, (((1,), (1,)), ((), ())),
        preferred_element_type=jnp.float32,
    )
    a = jnp.argmax(s, axis=1)
    onehot = (jax.lax.broadcasted_iota(jnp.int32, s.shape, 1)
              == a[:, None]).astype(jnp.bfloat16)
    sums_ref[...] += jax.lax.dot_general(
        onehot, rnb, (((0,), (0,)), ((), ())),
        preferred_element_type=jnp.float32,
    )
    counts_ref[...] += jax.lax.dot_general(
        onehot, jnp.ones((BLK, 1), jnp.bfloat16), (((0,), (0,)), ((), ())),
        preferred_element_type=jnp.float32,
    )

    @pl.when(i == NBLK - 1)
    def _fin():
        counts = counts_ref[...]
        means = sums_ref[...] / jnp.maximum(counts, 1.0)
        pn = pn_ref[...]
        upd = MOM * pn + (1.0 - MOM) * means
        updn = _norm_rows(upd)
        out_ref[...] = jnp.where(counts > 0.0, updn, pn)


def kernel(embeddings):
    emb = embeddings.astype(jnp.float32)
    return pl.pallas_call(
        _body,
        grid=(NBLK,),
        in_specs=[
            pl.BlockSpec((BANK, DIM), lambda i: (0, 0)),
            pl.BlockSpec((BLK, DIM), lambda i: (i + 1, 0)),
        ],
        out_specs=pl.BlockSpec((BANK, DIM), lambda i: (0, 0)),
        out_shape=jax.ShapeDtypeStruct((BANK, DIM), jnp.float32),
        scratch_shapes=[
            pltpu.VMEM((BANK, DIM), jnp.float32),
            pltpu.VMEM((BANK, DIM), jnp.float32),
            pltpu.VMEM((BANK, 1), jnp.float32),
        ],
    )(emb, emb)
